# baseline (speedup 1.0000x reference)
import functools

import jax
import jax.numpy as jnp
from jax.experimental import pallas as pl
from jax.experimental.pallas import tpu as pltpu

D_M = 64
KNN = 16
R = 256


def _pe_kernel(xyz_ref, xyzt_ref, hilo_ref, q_ref, x_ref, w1t_ref, b1_ref,
               w2t_ref, b2_ref, out_ref):
    n = xyz_ref.shape[1]

    ptst = xyzt_ref[0]
    hilo = hilo_ref[0]
    q = q_ref[0]

    d = None
    for c in range(3):
        t = (q[:, c:c + 1] - ptst[c:c + 1, :]) ** 2
        d = t if d is None else d + t

    inf = jnp.float32(jnp.inf)
    prev = jnp.full((R, 1), -jnp.inf, dtype=jnp.float32)
    deltas = []
    for _ in range(KNN):
        masked = jnp.where(d > prev, d, inf)
        prev = jnp.min(masked, axis=1, keepdims=True)
        sel = (masked == prev).astype(jnp.bfloat16)
        g2 = jnp.dot(sel, hilo, preferred_element_type=jnp.float32)
        deltas.append(q - (g2[:, :8] + g2[:, 8:]))
    delta = jnp.concatenate(deltas, axis=0)

    h = jnp.maximum(
        jnp.dot(delta, w1t_ref[...], preferred_element_type=jnp.float32)
        + b1_ref[...], 0.0)
    pe = (jnp.dot(h, w2t_ref[...], preferred_element_type=jnp.float32)
          + b2_ref[...])

    out_ref[0] = x_ref[0] + pe.reshape(KNN, R, D_M)


@jax.jit
def kernel(xyz, x, W1, b1, W2, b2):
    B, N, _ = xyz.shape
    pts = jnp.concatenate(
        [xyz, jnp.zeros((B, N, 5), dtype=xyz.dtype)], axis=-1)
    ptst = jnp.transpose(pts, (0, 2, 1))
    hi = pts.astype(jnp.bfloat16)
    lo = (pts - hi.astype(jnp.float32)).astype(jnp.bfloat16)
    hilo = jnp.concatenate([hi, lo], axis=-1)
    w1t = jnp.concatenate(
        [W1.T, jnp.zeros((5, D_M), dtype=W1.dtype)], axis=0)
    grid = (B, N // R)
    return pl.pallas_call(
        _pe_kernel,
        grid=grid,
        in_specs=[
            pl.BlockSpec((1, N, 8), lambda b, i: (b, 0, 0)),
            pl.BlockSpec((1, 8, N), lambda b, i: (b, 0, 0)),
            pl.BlockSpec((1, N, 16), lambda b, i: (b, 0, 0)),
            pl.BlockSpec((1, R, 8), lambda b, i: (b, i, 0)),
            pl.BlockSpec((1, KNN, R, D_M), lambda b, i: (b, 0, i, 0)),
            pl.BlockSpec((8, D_M), lambda b, i: (0, 0)),
            pl.BlockSpec((1, D_M), lambda b, i: (0, 0)),
            pl.BlockSpec((D_M, D_M), lambda b, i: (0, 0)),
            pl.BlockSpec((1, D_M), lambda b, i: (0, 0)),
        ],
        out_specs=pl.BlockSpec((1, KNN, R, D_M), lambda b, i: (b, 0, i, 0)),
        out_shape=jax.ShapeDtypeStruct(x.shape, x.dtype),
    )(pts, ptst, hilo, pts, x, w1t, b1.reshape(1, D_M), W2.T,
      b2.reshape(1, D_M))
